# baseline (speedup 1.0000x reference)
import functools

import jax
import jax.numpy as jnp
from jax import lax
from jax.experimental import pallas as pl
from jax.experimental.pallas import tpu as pltpu
from jax.experimental.pallas import tpu_sc as plsc

LANES = 16
NC = 2
NS = 16
NW = NC * NS


@functools.partial(jax.jit, static_argnums=(3, 4, 5))
def _embed_add(x, token_table, pe, S, D, B):
    CP = 16
    pos_per_w = S // NW
    n_steps = pos_per_w // CP
    mesh = plsc.VectorSubcoreMesh(core_axis_name="c", subcore_axis_name="s")

    @functools.partial(
        pl.kernel,
        mesh=mesh,
        out_type=jax.ShapeDtypeStruct((B, S, D), jnp.float32),
        scratch_types=[
            pltpu.VMEM((B, pos_per_w), jnp.int32),
            pltpu.VMEM((2, B * CP, D), jnp.float32),
            pltpu.VMEM((2 * CP, D), jnp.float32),
            pltpu.SemaphoreType.DMA,
            pltpu.SemaphoreType.DMA((2,)),
            pltpu.SemaphoreType.DMA((2,)),
            pltpu.SemaphoreType.DMA((2,)),
        ],
    )
    def k(x_hbm, table_hbm, pe_hbm, out_hbm,
          idx_v, rowsb, peb, isem, gsem, psem, ssem):
        wid = lax.axis_index("s") * NC + lax.axis_index("c")
        wpos = wid * pos_per_w

        pltpu.async_copy(x_hbm.at[:, pl.ds(wpos, pos_per_w)], idx_v,
                         isem).wait()

        def start_gathers(g):
            buf = g & 1
            for b in range(B):
                idx_sl = idx_v.at[b, pl.ds(g * CP, CP)]
                pltpu.async_copy(table_hbm.at[idx_sl],
                                 rowsb.at[buf, pl.ds(b * CP, CP)],
                                 gsem.at[buf])

        def start_pe(g):
            pb = g & 1
            pltpu.async_copy(pe_hbm.at[pl.ds(wpos + g * CP, CP)],
                             peb.at[pl.ds(pb * CP, CP)], psem.at[pb])

        def drain(sem_entry, nrows):
            pltpu.make_async_copy(pe_hbm.at[pl.ds(0, nrows)],
                                  rowsb.at[0, pl.ds(0, nrows)],
                                  sem_entry).wait()

        start_pe(0)
        start_gathers(0)

        def body(g, carry):
            buf = g & 1
            pb = g & 1

            @pl.when(g >= 1)
            def _():
                drain(ssem.at[(g + 1) & 1], B * CP)

            @pl.when(g < n_steps - 1)
            def _():
                start_gathers(g + 1)
                start_pe(g + 1)

            drain(gsem.at[buf], B * CP)
            drain(psem.at[pb], CP)

            pbase = pb * CP

            @plsc.parallel_loop(0, CP, unroll=2)
            def add_row(r):
                for v in range(D // LANES):
                    sl = pl.ds(v * LANES, LANES)
                    pv = peb[pbase + r, sl]
                    for b in range(B):
                        plsc.addupdate(rowsb.at[buf, b * CP + r, sl], pv)

            for b in range(B):
                pltpu.async_copy(rowsb.at[buf, pl.ds(b * CP, CP)],
                                 out_hbm.at[b, pl.ds(wpos + g * CP, CP)],
                                 ssem.at[buf])
            return carry

        lax.fori_loop(0, n_steps, body, 0)
        drain(ssem.at[(n_steps - 1) & 1], B * CP)

    return k(x, token_table, pe)


def kernel(x, token_table, pe):
    B, S = x.shape
    D = token_table.shape[1]
    return _embed_add(x.astype(jnp.int32), token_table, pe, S, D, B)
